# baseline (speedup 1.0000x reference)
import functools

import jax
import jax.numpy as jnp
from jax.experimental import pallas as pl
from jax.experimental.pallas import tpu as pltpu

BLK = 256
N_BF16_STEPS = 7


def _exp_rowsum_kernel(sims_ref, s0b_ref, r_ref):
    b = pl.program_id(0)
    e = jnp.exp(sims_ref[...] * jnp.float32(50.0))
    s0b_ref[...] = e.astype(jnp.bfloat16)
    r_ref[b, :] = jnp.float32(1.0) / jnp.sum(e, axis=1)


def _matvec_steps_kernel(r1_ref, s0b_ref, sims_ref, r_ref, c_ref, nblk):
    s = pl.program_id(0)
    b = pl.program_id(1)

    @pl.when((s == 0) & (b == 0))
    def _():
        r_ref[...] = r1_ref[...]

    is_col = (s % 2) == 0

    def col_pass(blk):
        rblk = r_ref[b, :].reshape(1, -1)
        part = jax.lax.dot_general(
            rblk, blk, (((1,), (0,)), ((), ())),
            preferred_element_type=jnp.float32,
            precision=jax.lax.Precision.HIGHEST,
        )

        @pl.when(b == 0)
        def _():
            c_ref[...] = part

        @pl.when(b != 0)
        def _():
            c_ref[...] = c_ref[...] + part

        @pl.when(b == nblk - 1)
        def _():
            c_ref[...] = jnp.float32(1.0) / c_ref[...]

    def row_pass(blk):
        rowsum = jax.lax.dot_general(
            blk, c_ref[...], (((1,), (1,)), ((), ())),
            preferred_element_type=jnp.float32,
            precision=jax.lax.Precision.HIGHEST,
        )
        r_ref[b, :] = jnp.float32(1.0) / rowsum[:, 0]

    @pl.when(s < N_BF16_STEPS)
    def _():
        blk = s0b_ref[...].astype(jnp.float32)

        @pl.when(is_col)
        def _():
            col_pass(blk)

        @pl.when(jnp.logical_not(is_col))
        def _():
            row_pass(blk)

    @pl.when(s >= N_BF16_STEPS)
    def _():
        blk = jnp.exp(sims_ref[...] * jnp.float32(50.0))

        @pl.when(is_col)
        def _():
            col_pass(blk)

        @pl.when(jnp.logical_not(is_col))
        def _():
            row_pass(blk)


def _finalize_kernel(sims_ref, r_ref, c_ref, s_ref, col_ref):
    b = pl.program_id(0)
    blk = jnp.exp(sims_ref[...] * jnp.float32(50.0))
    rblk = r_ref[b, :]
    sblk = blk * rblk[:, None] * c_ref[...]
    s_ref[...] = sblk
    col_ref[b, :] = jnp.argmax(sblk, axis=1).astype(jnp.int32)


def kernel(sims, batch_size):
    num_row, num_col = sims.shape
    nblk = num_row // BLK

    s0b, r1 = pl.pallas_call(
        _exp_rowsum_kernel,
        grid=(nblk,),
        in_specs=[pl.BlockSpec((BLK, num_col), lambda b: (b, 0))],
        out_specs=[
            pl.BlockSpec((BLK, num_col), lambda b: (b, 0)),
            pl.BlockSpec((nblk, BLK), lambda b: (0, 0)),
        ],
        out_shape=[
            jax.ShapeDtypeStruct((num_row, num_col), jnp.bfloat16),
            jax.ShapeDtypeStruct((nblk, BLK), jnp.float32),
        ],
    )(sims)

    r5, c5 = pl.pallas_call(
        functools.partial(_matvec_steps_kernel, nblk=nblk),
        grid=(9, nblk),
        in_specs=[
            pl.BlockSpec((nblk, BLK), lambda s, b: (0, 0)),
            pl.BlockSpec(
                (BLK, num_col),
                lambda s, b: (jnp.where(s < N_BF16_STEPS, b, nblk - 1), 0),
            ),
            pl.BlockSpec(
                (BLK, num_col),
                lambda s, b: (jnp.where(s >= N_BF16_STEPS, b, 0), 0),
            ),
        ],
        out_specs=[
            pl.BlockSpec((nblk, BLK), lambda s, b: (0, 0)),
            pl.BlockSpec((1, num_col), lambda s, b: (0, 0)),
        ],
        out_shape=[
            jax.ShapeDtypeStruct((nblk, BLK), jnp.float32),
            jax.ShapeDtypeStruct((1, num_col), jnp.float32),
        ],
    )(r1, s0b, sims)

    s, col = pl.pallas_call(
        _finalize_kernel,
        grid=(nblk,),
        in_specs=[
            pl.BlockSpec((BLK, num_col), lambda b: (b, 0)),
            pl.BlockSpec((nblk, BLK), lambda b: (0, 0)),
            pl.BlockSpec((1, num_col), lambda b: (0, 0)),
        ],
        out_specs=[
            pl.BlockSpec((BLK, num_col), lambda b: (b, 0)),
            pl.BlockSpec((nblk, BLK), lambda b: (0, 0)),
        ],
        out_shape=[
            jax.ShapeDtypeStruct((num_row, num_col), jnp.float32),
            jax.ShapeDtypeStruct((nblk, BLK), jnp.int32),
        ],
    )(sims, r5, c5)

    row = jnp.arange(num_row, dtype=jnp.int32)
    indices = jnp.stack((row, col.reshape(num_row)), axis=0)
    values = jnp.ones((num_row,), dtype=jnp.float32)
    return (s, indices, values)


# speedup vs baseline: 1.4330x; 1.4330x over previous
import functools

import jax
import jax.numpy as jnp
from jax.experimental import pallas as pl
from jax.experimental.pallas import tpu as pltpu

BLK1 = 256
BLK2 = 256
BLK3 = 256
N_BF16_STEPS = 7


def _exp_rowsum_kernel(sims_ref, s0b_ref, r_ref):
    b = pl.program_id(0)
    e = jnp.exp(sims_ref[...] * jnp.float32(50.0))
    s0b_ref[...] = e.astype(jnp.bfloat16)
    rowsum = jnp.sum(e, axis=1, keepdims=True)
    r_ref[pl.ds(b * BLK1, BLK1), :] = jnp.float32(1.0) / rowsum


def _matvec_steps_kernel(r1_ref, s0b_ref, sims_ref, r_ref, c_ref, nblk):
    s = pl.program_id(0)
    b = pl.program_id(1)

    @pl.when((s == 0) & (b == 0))
    def _():
        r_ref[...] = r1_ref[...]

    is_col = (s % 2) == 0

    def col_pass(blk):
        rblk = r_ref[pl.ds(b * BLK2, BLK2), :]
        part = jnp.sum(blk * rblk, axis=0, keepdims=True)

        @pl.when(b == 0)
        def _():
            c_ref[...] = part

        @pl.when(b != 0)
        def _():
            c_ref[...] = c_ref[...] + part

        @pl.when(b == nblk - 1)
        def _():
            c_ref[...] = jnp.float32(1.0) / c_ref[...]

    def row_pass(blk):
        rowsum = jnp.sum(blk * c_ref[...], axis=1, keepdims=True)
        r_ref[pl.ds(b * BLK2, BLK2), :] = jnp.float32(1.0) / rowsum

    @pl.when(s < N_BF16_STEPS)
    def _():
        blk = s0b_ref[...].astype(jnp.float32)

        @pl.when(is_col)
        def _():
            col_pass(blk)

        @pl.when(jnp.logical_not(is_col))
        def _():
            row_pass(blk)

    @pl.when(s >= N_BF16_STEPS)
    def _():
        blk = jnp.exp(sims_ref[...] * jnp.float32(50.0))

        @pl.when(is_col)
        def _():
            col_pass(blk)

        @pl.when(jnp.logical_not(is_col))
        def _():
            row_pass(blk)


def _finalize_kernel(sims_ref, r_ref, c_ref, s_ref, col_ref):
    b = pl.program_id(0)
    blk = jnp.exp(sims_ref[...] * jnp.float32(50.0))
    rblk = r_ref[pl.ds(b * BLK3, BLK3), :]
    sblk = blk * rblk * c_ref[...]
    s_ref[...] = sblk
    col_ref[pl.ds(b * BLK3, BLK3), :] = jnp.argmax(
        sblk, axis=1, keepdims=True
    ).astype(jnp.int32)


def kernel(sims, batch_size):
    num_row, num_col = sims.shape

    s0b, r1 = pl.pallas_call(
        _exp_rowsum_kernel,
        grid=(num_row // BLK1,),
        in_specs=[pl.BlockSpec((BLK1, num_col), lambda b: (b, 0))],
        out_specs=[
            pl.BlockSpec((BLK1, num_col), lambda b: (b, 0)),
            pl.BlockSpec((num_row, 1), lambda b: (0, 0)),
        ],
        out_shape=[
            jax.ShapeDtypeStruct((num_row, num_col), jnp.bfloat16),
            jax.ShapeDtypeStruct((num_row, 1), jnp.float32),
        ],
    )(sims)

    nblk2 = num_row // BLK2
    r5, c5 = pl.pallas_call(
        functools.partial(_matvec_steps_kernel, nblk=nblk2),
        grid=(9, nblk2),
        in_specs=[
            pl.BlockSpec((num_row, 1), lambda s, b: (0, 0)),
            pl.BlockSpec(
                (BLK2, num_col),
                lambda s, b: (jnp.where(s < N_BF16_STEPS, b, nblk2 - 1), 0),
            ),
            pl.BlockSpec(
                (BLK2, num_col),
                lambda s, b: (jnp.where(s >= N_BF16_STEPS, b, 0), 0),
            ),
        ],
        out_specs=[
            pl.BlockSpec((num_row, 1), lambda s, b: (0, 0)),
            pl.BlockSpec((1, num_col), lambda s, b: (0, 0)),
        ],
        out_shape=[
            jax.ShapeDtypeStruct((num_row, 1), jnp.float32),
            jax.ShapeDtypeStruct((1, num_col), jnp.float32),
        ],
    )(r1, s0b, sims)

    s, col = pl.pallas_call(
        _finalize_kernel,
        grid=(num_row // BLK3,),
        in_specs=[
            pl.BlockSpec((BLK3, num_col), lambda b: (b, 0)),
            pl.BlockSpec((num_row, 1), lambda b: (0, 0)),
            pl.BlockSpec((1, num_col), lambda b: (0, 0)),
        ],
        out_specs=[
            pl.BlockSpec((BLK3, num_col), lambda b: (b, 0)),
            pl.BlockSpec((num_row, 1), lambda b: (0, 0)),
        ],
        out_shape=[
            jax.ShapeDtypeStruct((num_row, num_col), jnp.float32),
            jax.ShapeDtypeStruct((num_row, 1), jnp.int32),
        ],
    )(sims, r5, c5)

    row = jnp.arange(num_row, dtype=jnp.int32)
    indices = jnp.stack((row, col.reshape(num_row)), axis=0)
    values = jnp.ones((num_row,), dtype=jnp.float32)
    return (s, indices, values)


# speedup vs baseline: 1.4681x; 1.0245x over previous
import functools

import jax
import jax.numpy as jnp
from jax.experimental import pallas as pl
from jax.experimental.pallas import tpu as pltpu

BLK1 = 256
BLK2 = 256
BLK3 = 256
N_BF16_STEPS = 7


def _exp_rowsum_kernel(sims_ref, s0b_ref, r_ref):
    b = pl.program_id(0)
    e = jnp.exp(sims_ref[...] * jnp.float32(50.0))
    s0b_ref[...] = e.astype(jnp.bfloat16)
    rowsum = jnp.sum(e, axis=1, keepdims=True)
    r_ref[pl.ds(b * BLK1, BLK1), :] = jnp.float32(1.0) / rowsum


def _matvec_steps_kernel(r1_ref, s0b_ref, sims_ref, r_ref, c_ref, nblk):
    s = pl.program_id(0)
    b = pl.program_id(1)

    @pl.when((s == 0) & (b == 0))
    def _():
        r_ref[...] = r1_ref[...]

    is_col = (s % 2) == 0

    def col_pass(blk):
        rblk = r_ref[pl.ds(b * BLK2, BLK2), :]
        part = jnp.sum(blk * rblk, axis=0, keepdims=True)

        @pl.when(b == 0)
        def _():
            c_ref[...] = part

        @pl.when(b != 0)
        def _():
            c_ref[...] = c_ref[...] + part

        @pl.when(b == nblk - 1)
        def _():
            c_ref[...] = jnp.float32(1.0) / c_ref[...]

    def row_pass(blk):
        rowsum = jnp.sum(blk * c_ref[...], axis=1, keepdims=True)
        r_ref[pl.ds(b * BLK2, BLK2), :] = jnp.float32(1.0) / rowsum

    @pl.when((s < N_BF16_STEPS) & is_col)
    def _():
        col_pass(s0b_ref[...].astype(jnp.float32))

    @pl.when((s < N_BF16_STEPS) & jnp.logical_not(is_col))
    def _():
        row_pass(s0b_ref[...].astype(jnp.float32))

    @pl.when((s >= N_BF16_STEPS) & is_col)
    def _():
        col_pass(jnp.exp(sims_ref[...] * jnp.float32(50.0)))

    @pl.when((s >= N_BF16_STEPS) & jnp.logical_not(is_col))
    def _():
        row_pass(jnp.exp(sims_ref[...] * jnp.float32(50.0)))


def _finalize_kernel(sims_ref, r_ref, c_ref, s_ref, col_ref):
    b = pl.program_id(0)
    blk = jnp.exp(sims_ref[...] * jnp.float32(50.0))
    rblk = r_ref[pl.ds(b * BLK3, BLK3), :]
    sblk = blk * rblk * c_ref[...]
    s_ref[...] = sblk
    col_ref[pl.ds(b * BLK3, BLK3), :] = jnp.argmax(
        sblk, axis=1, keepdims=True
    ).astype(jnp.int32)


def kernel(sims, batch_size):
    num_row, num_col = sims.shape

    s0b, r1 = pl.pallas_call(
        _exp_rowsum_kernel,
        grid=(num_row // BLK1,),
        in_specs=[pl.BlockSpec((BLK1, num_col), lambda b: (b, 0))],
        out_specs=[
            pl.BlockSpec((BLK1, num_col), lambda b: (b, 0)),
            pl.BlockSpec((num_row, 1), lambda b: (0, 0)),
        ],
        out_shape=[
            jax.ShapeDtypeStruct((num_row, num_col), jnp.bfloat16),
            jax.ShapeDtypeStruct((num_row, 1), jnp.float32),
        ],
    )(sims)

    nblk2 = num_row // BLK2
    r5, c5 = pl.pallas_call(
        functools.partial(_matvec_steps_kernel, nblk=nblk2),
        grid=(9, nblk2),
        in_specs=[
            pl.BlockSpec((num_row, 1), lambda s, b: (0, 0)),
            pl.BlockSpec(
                (BLK2, num_col),
                lambda s, b: (jnp.where(s < N_BF16_STEPS, b, nblk2 - 1), 0),
            ),
            pl.BlockSpec(
                (BLK2, num_col),
                lambda s, b: (jnp.where(s >= N_BF16_STEPS, b, 0), 0),
            ),
        ],
        out_specs=[
            pl.BlockSpec((num_row, 1), lambda s, b: (0, 0)),
            pl.BlockSpec((1, num_col), lambda s, b: (0, 0)),
        ],
        out_shape=[
            jax.ShapeDtypeStruct((num_row, 1), jnp.float32),
            jax.ShapeDtypeStruct((1, num_col), jnp.float32),
        ],
    )(r1, s0b, sims)

    s, col = pl.pallas_call(
        _finalize_kernel,
        grid=(num_row // BLK3,),
        in_specs=[
            pl.BlockSpec((BLK3, num_col), lambda b: (b, 0)),
            pl.BlockSpec((num_row, 1), lambda b: (0, 0)),
            pl.BlockSpec((1, num_col), lambda b: (0, 0)),
        ],
        out_specs=[
            pl.BlockSpec((BLK3, num_col), lambda b: (b, 0)),
            pl.BlockSpec((num_row, 1), lambda b: (0, 0)),
        ],
        out_shape=[
            jax.ShapeDtypeStruct((num_row, num_col), jnp.float32),
            jax.ShapeDtypeStruct((num_row, 1), jnp.int32),
        ],
    )(sims, r5, c5)

    row = jnp.arange(num_row, dtype=jnp.int32)
    indices = jnp.stack((row, col.reshape(num_row)), axis=0)
    values = jnp.ones((num_row,), dtype=jnp.float32)
    return (s, indices, values)
